# baseline (speedup 1.0000x reference)
import functools

import jax
import jax.numpy as jnp
from jax import lax
from jax.experimental import pallas as pl
from jax.experimental.pallas import tpu as pltpu
from jax.experimental.pallas import tpu_sc as plsc

NUM_CORES = 2
NUM_SUBCORES = 16
NW = NUM_CORES * NUM_SUBCORES

IN_ROWS = 8192
OUT_ROWS = 32768
D = 1024
ROWS_PER_W = OUT_ROWS // NW
CHUNK = 64
N_CHUNKS = ROWS_PER_W // CHUNK


def _gather_body(feeds_hbm, idx_hbm, out_hbm, idx_v, rows_v, sem):
    wid = lax.axis_index("s") * NUM_CORES + lax.axis_index("c")
    base = wid * ROWS_PER_W

    def chunk(c, carry):
        start = base + c * CHUNK
        pltpu.sync_copy(idx_hbm.at[pl.ds(start, CHUNK)], idx_v)
        pltpu.async_copy(feeds_hbm.at[idx_v], rows_v, sem).wait()
        pltpu.sync_copy(rows_v, out_hbm.at[pl.ds(start, CHUNK)])
        return carry

    lax.fori_loop(0, N_CHUNKS, chunk, 0)


_gather = functools.partial(
    pl.kernel,
    out_type=jax.ShapeDtypeStruct((OUT_ROWS, D), jnp.float32),
    mesh=plsc.VectorSubcoreMesh(core_axis_name="c", subcore_axis_name="s"),
    scratch_types=[
        pltpu.VMEM((CHUNK,), jnp.int32),
        pltpu.VMEM((CHUNK, D), jnp.float32),
        pltpu.SemaphoreType.DMA,
    ],
)(_gather_body)


def kernel(feeds, feeds_repeat_times, output_feeds_size):
    rt = feeds_repeat_times.astype(jnp.int32)
    cumulative = jnp.cumsum(rt)
    positions = jnp.arange(OUT_ROWS, dtype=jnp.int32)
    src = jnp.searchsorted(cumulative, positions, side="right").astype(jnp.int32)
    src = jnp.where(positions < output_feeds_size, src, IN_ROWS)
    feeds_ext = jnp.concatenate(
        [feeds, jnp.zeros((8, D), feeds.dtype)], axis=0)
    return _gather(feeds_ext, src)


# speedup vs baseline: 4.0806x; 4.0806x over previous
import functools

import jax
import jax.numpy as jnp
from jax import lax
from jax.experimental import pallas as pl
from jax.experimental.pallas import tpu as pltpu
from jax.experimental.pallas import tpu_sc as plsc

NUM_CORES = 2
NUM_SUBCORES = 16
NW = NUM_CORES * NUM_SUBCORES
L = 16

IN_ROWS = 8192
OUT_ROWS = 32768
D = 1024
ROWS_PER_W = OUT_ROWS // NW
CHUNK = 32
N_CHUNKS = ROWS_PER_W // CHUNK
MAX_REP = 3


def _body(feeds_hbm, rt_hbm, cum_hbm, lim_hbm, out_hbm,
          rt_v, cum_v, idx_v, lim_v, rows0, rows1, sem0, sem1):
    wid = lax.axis_index("s") * NUM_CORES + lax.axis_index("c")
    base = wid * ROWS_PER_W

    pltpu.sync_copy(rt_hbm, rt_v)
    pltpu.sync_copy(cum_hbm, cum_v)
    pltpu.sync_copy(lim_hbm, lim_v)
    limit = lim_v[...]

    def init(j, carry):
        idx_v[pl.ds(j * L, L)] = jnp.full((L,), IN_ROWS, jnp.int32)
        return carry

    lax.fori_loop(0, ROWS_PER_W // L, init, 0)

    lane = lax.iota(jnp.int32, L)

    def scan(j, carry):
        r = rt_v[pl.ds(j * L, L)]
        off = cum_v[pl.ds(j * L, L)] - r
        rowid = j * L + lane
        for k in range(MAX_REP):
            gpos = off + k
            pos = gpos - base
            mask = (r > k) & (pos >= 0) & (pos < ROWS_PER_W) & (gpos < limit)
            plsc.store_scatter(idx_v, [pos], rowid, mask=mask)
        return carry

    lax.fori_loop(0, IN_ROWS // L, scan, 0)

    rows = (rows0, rows1)
    sems = (sem0, sem1)

    def gather_desc(c, b):
        return pltpu.make_async_copy(
            feeds_hbm.at[idx_v.at[pl.ds(c * CHUNK, CHUNK)]], rows[b], sems[b])

    gather_desc(0, 0).start()
    gather_desc(1, 1).start()

    def pair(p, carry):
        for b in range(2):
            c = p * 2 + b
            gather_desc(c, b).wait()
            pltpu.sync_copy(rows[b], out_hbm.at[pl.ds(base + c * CHUNK, CHUNK)])
            cnext = jnp.minimum(c + 2, N_CHUNKS - 1)
            gather_desc(cnext, b).start()
        return carry

    lax.fori_loop(0, N_CHUNKS // 2, pair, 0)
    gather_desc(N_CHUNKS - 1, 0).wait()
    gather_desc(N_CHUNKS - 1, 1).wait()


_sc_repeat = functools.partial(
    pl.kernel,
    out_type=jax.ShapeDtypeStruct((OUT_ROWS, D), jnp.float32),
    mesh=plsc.VectorSubcoreMesh(core_axis_name="c", subcore_axis_name="s"),
    compiler_params=pltpu.CompilerParams(needs_layout_passes=False),
    scratch_types=[
        pltpu.VMEM((IN_ROWS,), jnp.int32),
        pltpu.VMEM((IN_ROWS,), jnp.int32),
        pltpu.VMEM((ROWS_PER_W,), jnp.int32),
        pltpu.VMEM((L,), jnp.int32),
        pltpu.VMEM((CHUNK, D), jnp.float32),
        pltpu.VMEM((CHUNK, D), jnp.float32),
        pltpu.SemaphoreType.DMA,
        pltpu.SemaphoreType.DMA,
    ],
)(_body)


def kernel(feeds, feeds_repeat_times, output_feeds_size):
    rt = feeds_repeat_times.astype(jnp.int32)
    cum = jnp.cumsum(rt)
    limit = jnp.full((L,), jnp.minimum(output_feeds_size, OUT_ROWS), jnp.int32)
    feeds_ext = jnp.concatenate([feeds, jnp.zeros((8, D), feeds.dtype)], axis=0)
    return _sc_repeat(feeds_ext, rt, cum, limit)
